# baseline (speedup 1.0000x reference)
import functools

import jax
import jax.numpy as jnp
from jax.experimental import pallas as pl
from jax.experimental.pallas import tpu as pltpu

_TM = 512
_LANES = 128
_GRP = 16
_NEG = -1e30


def _gelu(h):
    return 0.5 * h * (1.0 + jax.lax.erf(h * 0.7071067811865476))


def _ln_f32(h, g, b):
    m = jnp.mean(h, axis=-1, keepdims=True)
    d = h - m
    v = jnp.mean(d * d, axis=-1, keepdims=True)
    return d * jax.lax.rsqrt(v + 1e-5) * g + b


def _top2_t(p):
    idx = jax.lax.broadcasted_iota(jnp.int32, p.shape, 0).astype(jnp.float32)
    w1 = jnp.max(p, axis=0, keepdims=True)
    i1 = jnp.min(jnp.where(p == w1, idx, jnp.float32(1e9)), axis=0, keepdims=True)
    p2 = jnp.where(idx == i1, _NEG, p)
    w2 = jnp.max(p2, axis=0, keepdims=True)
    i2 = jnp.min(jnp.where(p2 == w2, idx, jnp.float32(1e9)), axis=0, keepdims=True)
    return w1, i1, w2, i2


def _softmax_t(l):
    m = jnp.max(l, axis=0, keepdims=True)
    e = jnp.exp(l - m)
    return e / jnp.sum(e, axis=0, keepdims=True)


def _pair_out(a, b):
    return jnp.transpose(jnp.concatenate([a, b], axis=0), (1, 0))


def _router_body(C, H, AH,
                 x_ref, cW1_ref, eW1_ref, aW1_ref,
                 cW2_ref, eW2_ref, aW2_ref, P_ref,
                 cw_ref, ci_ref, ew0_ref, ei0_ref, ew1_ref, ei1_ref, ad_ref,
                 W1s, cW2s, W2es, aW2s, b2s):
    i = pl.program_id(0)
    L = _LANES
    f32 = jnp.float32
    bf16 = jnp.bfloat16

    @pl.when(i == 0)
    def _():
        W1s[:, 0:H] = cW1_ref[...].astype(bf16)
        for c in range(C):
            W1s[:, H + c * H:H + (c + 1) * H] = eW1_ref[c * x_ref.shape[1]:(c + 1) * x_ref.shape[1], :].astype(bf16)
        W1s[:, H + C * H:H + C * H + AH] = aW1_ref[...].astype(bf16)

        nc = cW2_ref.shape[1]
        cW2s[...] = jnp.concatenate(
            [cW2_ref[...].astype(bf16), jnp.zeros((H, L - nc), bf16)], axis=1)
        ne = eW2_ref.shape[1]
        eW2f = eW2_ref[...]
        for c in range(C):
            parts = []
            if c > 0:
                parts.append(jnp.zeros((H, _GRP * c), bf16))
            parts.append(eW2f[c * H:(c + 1) * H, :].astype(bf16))
            parts.append(jnp.zeros((H, L - _GRP * c - ne), bf16))
            W2es[c * H:(c + 1) * H, :] = jnp.concatenate(parts, axis=1)
        aW2s[...] = jnp.concatenate(
            [aW2_ref[...].astype(bf16), jnp.zeros((AH, L - 1), bf16)], axis=1)

        b2s[0:1, :] = jnp.concatenate(
            [P_ref[30:31, 0:nc], jnp.full((1, L - nc), _NEG, f32)], axis=1)
        eb2f = P_ref[31:32, 0:C * ne]
        erow = jnp.full((1, L), _NEG, f32)
        lane = jax.lax.broadcasted_iota(jnp.int32, (1, L), 1)
        for c in range(C):
            parts = []
            if c > 0:
                parts.append(jnp.zeros((1, _GRP * c), f32))
            parts.append(eb2f[:, c * ne:(c + 1) * ne])
            parts.append(jnp.zeros((1, L - _GRP * c - ne), f32))
            row = jnp.concatenate(parts, axis=1)
            m = (lane >= _GRP * c) & (lane < _GRP * c + ne)
            erow = jnp.where(m, row, erow)
        b2s[1:2, :] = erow
        b2s[2:3, :] = jnp.concatenate(
            [P_ref[32:33, 0:1], jnp.zeros((1, L - 1), f32)], axis=1)

    xb = x_ref[...].astype(bf16)

    mm = jnp.dot(xb, W1s[...], preferred_element_type=jnp.float32)

    h = _gelu(_ln_f32(mm[:, :H] + P_ref[0:1, :], P_ref[1:2, :], P_ref[2:3, :]))
    cl = jnp.dot(h.astype(jnp.bfloat16), cW2s[...],
                 preferred_element_type=jnp.float32) + b2s[0:1, :]
    clT = jnp.transpose(cl, (1, 0))[:C]
    cp = _softmax_t(clT)
    cw1, ci1, cw2, ci2 = _top2_t(cp)

    ehb = []
    for c in range(C):
        seg = mm[:, H + c * H:H + (c + 1) * H]
        eh = _gelu(_ln_f32(seg + P_ref[3 + c:4 + c, :], P_ref[11 + c:12 + c, :], P_ref[19 + c:20 + c, :]))
        ehb.append(eh.astype(jnp.bfloat16))
    ehb = jnp.concatenate(ehb, axis=1)
    el = jnp.dot(ehb, W2es[...],
                 preferred_element_type=jnp.float32) + b2s[1:2, :]
    elT = jnp.transpose(el, (1, 0))

    sel0 = jnp.full((_GRP, elT.shape[1]), _NEG, dtype=jnp.float32)
    sel1 = jnp.full((_GRP, elT.shape[1]), _NEG, dtype=jnp.float32)
    for c in range(C):
        grp = elT[c * _GRP:(c + 1) * _GRP]
        sel0 = jnp.where(ci1 == c, grp, sel0)
        sel1 = jnp.where(ci2 == c, grp, sel1)
    e0w1, e0i1, e0w2, e0i2 = _top2_t(_softmax_t(sel0))
    e1w1, e1i1, e1w2, e1i2 = _top2_t(_softmax_t(sel1))

    ah = _gelu(_ln_f32(mm[:, H + C * H:H + C * H + AH] + P_ref[27:28, :AH],
                       P_ref[28:29, :AH], P_ref[29:30, :AH]))
    av = jnp.dot(ah.astype(jnp.bfloat16), aW2s[...],
                 preferred_element_type=jnp.float32) + b2s[2:3, :]
    ad = jax.nn.sigmoid(av[:, 0:1])

    cw_ref[...] = _pair_out(cw1, cw2)
    ci_ref[...] = _pair_out(ci1, ci2).astype(jnp.int32)
    ew0_ref[...] = _pair_out(e0w1, e0w2)
    ei0_ref[...] = _pair_out(e0i1, e0i2).astype(jnp.int32)
    ew1_ref[...] = _pair_out(e1w1, e1w2)
    ei1_ref[...] = _pair_out(e1i1, e1i2).astype(jnp.int32)
    ad_ref[...] = ad


def kernel(x, cW1, cb1, cg, cbb, cW2, cb2, eW1, eb1, eg, ebb, eW2, eb2,
           aW1, ab1, ag, abb, aW2, ab2):
    Bz, Sz, Dz = x.shape
    T = Bz * Sz
    C = eW1.shape[0]
    H = cW1.shape[1]
    AH = aW1.shape[1]
    EPC = eW2.shape[2]
    L = _LANES

    xf = x.reshape(T, Dz)
    eW1f = eW1.reshape(C * Dz, H)
    eW2f = eW2.reshape(C * H, EPC)

    def row(v):
        v = v.reshape(1, -1)
        return jnp.pad(v, ((0, 0), (0, H - v.shape[1])))
    P = jnp.concatenate(
        [row(cb1), row(cg), row(cbb),
         eb1, eg, ebb,
         row(ab1), row(ag), row(abb),
         row(cb2), row(eb2.reshape(-1)), row(ab2)], axis=0)

    grid = (T // _TM,)
    tok_spec = pl.BlockSpec((_TM, Dz), lambda i: (i, 0))
    full = lambda *shape: pl.BlockSpec(shape, lambda i: (0,) * len(shape))
    out2 = pl.BlockSpec((_TM, 2), lambda i: (i, 0))
    out1 = pl.BlockSpec((_TM, 1), lambda i: (i, 0))

    f32 = jnp.float32
    i32 = jnp.int32
    outs = (
        jax.ShapeDtypeStruct((T, 2), f32), jax.ShapeDtypeStruct((T, 2), i32),
        jax.ShapeDtypeStruct((T, 2), f32), jax.ShapeDtypeStruct((T, 2), i32),
        jax.ShapeDtypeStruct((T, 2), f32), jax.ShapeDtypeStruct((T, 2), i32),
        jax.ShapeDtypeStruct((T, 1), f32),
    )

    cw, ci, ew0, ei0, ew1, ei1, ad = pl.pallas_call(
        functools.partial(_router_body, C, H, AH),
        grid=grid,
        in_specs=[
            tok_spec,
            full(Dz, H), full(C * Dz, H), full(Dz, AH),
            full(H, C), full(C * H, EPC), full(AH, 1), full(33, H),
        ],
        out_specs=[out2, out2, out2, out2, out2, out2, out1],
        out_shape=outs,
        scratch_shapes=[
            pltpu.VMEM((Dz, H + C * H + AH), jnp.bfloat16),
            pltpu.VMEM((H, L), jnp.bfloat16),
            pltpu.VMEM((C * H, L), jnp.bfloat16),
            pltpu.VMEM((AH, L), jnp.bfloat16),
            pltpu.VMEM((8, L), jnp.float32),
        ],
        compiler_params=pltpu.CompilerParams(
            dimension_semantics=("arbitrary",),
        ),
    )(xf, cW1, eW1f, aW1,
      cW2, eW2f, aW2, P)

    return (cw, ci, ew0, ei0, ew1, ei1, ad.reshape(Bz, Sz, 1))


# speedup vs baseline: 1.1089x; 1.1089x over previous
import functools

import jax
import jax.numpy as jnp
from jax.experimental import pallas as pl
from jax.experimental.pallas import tpu as pltpu

_TM = 512
_LANES = 128
_GRP = 16
_NEG = -1e30


def _gelu(h):
    return 0.5 * h * (1.0 + jax.lax.erf(h * 0.7071067811865476))


def _ln_f32(h, g, b):
    m = jnp.mean(h, axis=-1, keepdims=True)
    d = h - m
    v = jnp.mean(d * d, axis=-1, keepdims=True)
    return d * jax.lax.rsqrt(v + 1e-5) * g + b


def _top2_t(p):
    idx = jax.lax.broadcasted_iota(jnp.int32, p.shape, 0).astype(jnp.float32)
    w1 = jnp.max(p, axis=0, keepdims=True)
    i1 = jnp.min(jnp.where(p == w1, idx, jnp.float32(1e9)), axis=0, keepdims=True)
    p2 = jnp.where(idx == i1, _NEG, p)
    w2 = jnp.max(p2, axis=0, keepdims=True)
    i2 = jnp.min(jnp.where(p2 == w2, idx, jnp.float32(1e9)), axis=0, keepdims=True)
    return w1, i1, w2, i2


def _softmax_t(l):
    m = jnp.max(l, axis=0, keepdims=True)
    e = jnp.exp(l - m)
    return e / jnp.sum(e, axis=0, keepdims=True)


def _pair_out(a, b):
    return jnp.transpose(jnp.concatenate([a, b], axis=0), (1, 0))


def _router_body(C, H, AH,
                 x_ref,
                 cW1_h, cb1_h, cg_h, cbb_h, cW2_h, cb2_h,
                 eW1_h, eb1_h, eg_h, ebb_h, eW2_h, eb2_h,
                 aW1_h, ab1_h, ag_h, abb_h, aW2_h, ab2_h,
                 cw_ref, ci_ref, ew0_ref, ei0_ref, ew1_ref, ei1_ref, ad_ref,
                 W1s, cW2s, W2es, aW2s, b2s,
                 cW1t, cb1t, cgt, cbbt, cW2t, cb2t,
                 eW1t, eb1t, egt, ebbt, eW2t, eb2t,
                 aW1t, ab1t, agt, abbt, aW2t, ab2t,
                 sems):
    i = pl.program_id(0)
    L = _LANES
    D = x_ref.shape[1]
    f32 = jnp.float32
    bf16 = jnp.bfloat16

    @pl.when(i == 0)
    def _():
        srcs = (cW1_h, cb1_h, cg_h, cbb_h, cW2_h, cb2_h,
                eW1_h, eb1_h, eg_h, ebb_h, eW2_h, eb2_h,
                aW1_h, ab1_h, ag_h, abb_h, aW2_h, ab2_h)
        dsts = (cW1t, cb1t, cgt, cbbt, cW2t, cb2t,
                eW1t, eb1t, egt, ebbt, eW2t, eb2t,
                aW1t, ab1t, agt, abbt, aW2t, ab2t)
        copies = [pltpu.make_async_copy(s, d, sems.at[k])
                  for k, (s, d) in enumerate(zip(srcs, dsts))]
        for cp in copies:
            cp.start()
        for cp in copies:
            cp.wait()

        W1s[:, 0:H] = cW1t[...].astype(bf16)
        for c in range(C):
            W1s[:, H + c * H:H + (c + 1) * H] = eW1t[c * D:(c + 1) * D, :].astype(bf16)
        W1s[:, H + C * H:H + C * H + AH] = aW1t[...].astype(bf16)

        nc = cW2t.shape[1]
        cW2s[...] = jnp.concatenate(
            [cW2t[...].astype(bf16), jnp.zeros((H, L - nc), bf16)], axis=1)
        ne = eW2t.shape[1]
        for c in range(C):
            parts = []
            if c > 0:
                parts.append(jnp.zeros((H, _GRP * c), bf16))
            parts.append(eW2t[c * H:(c + 1) * H, :].astype(bf16))
            parts.append(jnp.zeros((H, L - _GRP * c - ne), bf16))
            W2es[c * H:(c + 1) * H, :] = jnp.concatenate(parts, axis=1)
        aW2s[...] = jnp.concatenate(
            [aW2t[...].astype(bf16), jnp.zeros((AH, L - 1), bf16)], axis=1)

        b2s[0:1, :] = jnp.concatenate(
            [cb2t[...], jnp.full((1, L - nc), _NEG, f32)], axis=1)
        erow = jnp.full((1, L), _NEG, f32)
        lane = jax.lax.broadcasted_iota(jnp.int32, (1, L), 1)
        for c in range(C):
            parts = []
            if c > 0:
                parts.append(jnp.zeros((1, _GRP * c), f32))
            parts.append(eb2t[c:c + 1, :])
            parts.append(jnp.zeros((1, L - _GRP * c - ne), f32))
            row = jnp.concatenate(parts, axis=1)
            m = (lane >= _GRP * c) & (lane < _GRP * c + ne)
            erow = jnp.where(m, row, erow)
        b2s[1:2, :] = erow
        b2s[2:3, :] = jnp.concatenate(
            [ab2t[...], jnp.zeros((1, L - 1), f32)], axis=1)

    xb = x_ref[...].astype(bf16)

    mm = jnp.dot(xb, W1s[...], preferred_element_type=jnp.float32)

    h = _gelu(_ln_f32(mm[:, :H] + cb1t[...], cgt[...], cbbt[...]))
    cl = jnp.dot(h.astype(jnp.bfloat16), cW2s[...],
                 preferred_element_type=jnp.float32) + b2s[0:1, :]
    clT = jnp.transpose(cl, (1, 0))[:C]
    cp_ = _softmax_t(clT)
    cw1, ci1, cw2, ci2 = _top2_t(cp_)

    ehb = []
    for c in range(C):
        seg = mm[:, H + c * H:H + (c + 1) * H]
        eh = _gelu(_ln_f32(seg + eb1t[c:c + 1, :], egt[c:c + 1, :],
                           ebbt[c:c + 1, :]))
        ehb.append(eh.astype(jnp.bfloat16))
    ehb = jnp.concatenate(ehb, axis=1)
    el = jnp.dot(ehb, W2es[...],
                 preferred_element_type=jnp.float32) + b2s[1:2, :]
    elT = jnp.transpose(el, (1, 0))

    sel0 = jnp.full((_GRP, elT.shape[1]), _NEG, dtype=jnp.float32)
    sel1 = jnp.full((_GRP, elT.shape[1]), _NEG, dtype=jnp.float32)
    for c in range(C):
        grp = elT[c * _GRP:(c + 1) * _GRP]
        sel0 = jnp.where(ci1 == c, grp, sel0)
        sel1 = jnp.where(ci2 == c, grp, sel1)
    e0w1, e0i1, e0w2, e0i2 = _top2_t(_softmax_t(sel0))
    e1w1, e1i1, e1w2, e1i2 = _top2_t(_softmax_t(sel1))

    ah = _gelu(_ln_f32(mm[:, H + C * H:H + C * H + AH] + ab1t[...],
                       agt[...], abbt[...]))
    av = jnp.dot(ah.astype(jnp.bfloat16), aW2s[...],
                 preferred_element_type=jnp.float32) + b2s[2:3, :]
    ad = jax.nn.sigmoid(av[:, 0:1])

    cw_ref[...] = _pair_out(cw1, cw2)
    ci_ref[...] = _pair_out(ci1, ci2).astype(jnp.int32)
    ew0_ref[...] = _pair_out(e0w1, e0w2)
    ei0_ref[...] = _pair_out(e0i1, e0i2).astype(jnp.int32)
    ew1_ref[...] = _pair_out(e1w1, e1w2)
    ei1_ref[...] = _pair_out(e1i1, e1i2).astype(jnp.int32)
    ad_ref[...] = ad


def kernel(x, cW1, cb1, cg, cbb, cW2, cb2, eW1, eb1, eg, ebb, eW2, eb2,
           aW1, ab1, ag, abb, aW2, ab2):
    Bz, Sz, Dz = x.shape
    T = Bz * Sz
    C = eW1.shape[0]
    H = cW1.shape[1]
    AH = aW1.shape[1]
    EPC = eW2.shape[2]
    L = _LANES

    xf = x.reshape(T, Dz)
    eW1f = eW1.reshape(C * Dz, H)
    eW2f = eW2.reshape(C * H, EPC)
    cb1r, cgr, cbbr = cb1.reshape(1, H), cg.reshape(1, H), cbb.reshape(1, H)
    ab1r, agr, abbr = ab1.reshape(1, AH), ag.reshape(1, AH), abb.reshape(1, AH)
    cb2r = cb2.reshape(1, C)
    ab2r = ab2.reshape(1, 1)

    grid = (T // _TM,)
    tok_spec = pl.BlockSpec((_TM, Dz), lambda i: (i, 0))
    anyspec = pl.BlockSpec(memory_space=pl.ANY)
    out2 = pl.BlockSpec((_TM, 2), lambda i: (i, 0))
    out1 = pl.BlockSpec((_TM, 1), lambda i: (i, 0))

    f32 = jnp.float32
    i32 = jnp.int32
    outs = (
        jax.ShapeDtypeStruct((T, 2), f32), jax.ShapeDtypeStruct((T, 2), i32),
        jax.ShapeDtypeStruct((T, 2), f32), jax.ShapeDtypeStruct((T, 2), i32),
        jax.ShapeDtypeStruct((T, 2), f32), jax.ShapeDtypeStruct((T, 2), i32),
        jax.ShapeDtypeStruct((T, 1), f32),
    )
    V = pltpu.VMEM

    cw, ci, ew0, ei0, ew1, ei1, ad = pl.pallas_call(
        functools.partial(_router_body, C, H, AH),
        grid=grid,
        in_specs=[tok_spec] + [anyspec] * 18,
        out_specs=[out2, out2, out2, out2, out2, out2, out1],
        out_shape=outs,
        scratch_shapes=[
            V((Dz, H + C * H + AH), jnp.bfloat16),
            V((H, L), jnp.bfloat16),
            V((C * H, L), jnp.bfloat16),
            V((AH, L), jnp.bfloat16),
            V((8, L), f32),
            V((Dz, H), f32), V((1, H), f32), V((1, H), f32), V((1, H), f32),
            V((H, C), f32), V((1, C), f32),
            V((C * Dz, H), f32), V((C, H), f32), V((C, H), f32), V((C, H), f32),
            V((C * H, EPC), f32), V((C, EPC), f32),
            V((Dz, AH), f32), V((1, AH), f32), V((1, AH), f32), V((1, AH), f32),
            V((AH, 1), f32), V((1, 1), f32),
            pltpu.SemaphoreType.DMA((18,)),
        ],
        compiler_params=pltpu.CompilerParams(
            dimension_semantics=("arbitrary",),
        ),
    )(xf,
      cW1, cb1r, cgr, cbbr, cW2, cb2r,
      eW1f, eb1, eg, ebb, eW2f, eb2,
      aW1, ab1r, agr, abbr, aW2, ab2r)

    return (cw, ci, ew0, ei0, ew1, ei1, ad.reshape(Bz, Sz, 1))


# speedup vs baseline: 1.4262x; 1.2862x over previous
import functools

import jax
import jax.numpy as jnp
from jax.experimental import pallas as pl
from jax.experimental.pallas import tpu as pltpu

_TM = 512
_LANES = 128
_GRP = 16
_NEG = -1e30


def _gelu(h):
    return 0.5 * h * (1.0 + jax.lax.erf(h * 0.7071067811865476))


def _ln_f32(h, g, b):
    m = jnp.mean(h, axis=-1, keepdims=True)
    d = h - m
    v = jnp.mean(d * d, axis=-1, keepdims=True)
    return d * jax.lax.rsqrt(v + 1e-5) * g + b


def _top2_t(p):
    idx = jax.lax.broadcasted_iota(jnp.int32, p.shape, 0).astype(jnp.float32)
    w1 = jnp.max(p, axis=0, keepdims=True)
    i1 = jnp.min(jnp.where(p == w1, idx, jnp.float32(1e9)), axis=0, keepdims=True)
    p2 = jnp.where(idx == i1, _NEG, p)
    w2 = jnp.max(p2, axis=0, keepdims=True)
    i2 = jnp.min(jnp.where(p2 == w2, idx, jnp.float32(1e9)), axis=0, keepdims=True)
    return w1, i1, w2, i2


def _softmax_t(l):
    m = jnp.max(l, axis=0, keepdims=True)
    e = jnp.exp(l - m)
    return e / jnp.sum(e, axis=0, keepdims=True)


def _pair_out(a, b):
    return jnp.transpose(jnp.concatenate([a, b], axis=0), (1, 0))


def _router_body(C, H, AH,
                 x_ref,
                 cW1_h, cb1_h, cg_h, cbb_h, cW2_h, cb2_h,
                 eW1_h, eb1_h, eg_h, ebb_h, eW2_h, eb2_h,
                 aW1_h, ab1_h, ag_h, abb_h, aW2_h, ab2_h,
                 cw_ref, ci_ref, ew0_ref, ei0_ref, ew1_ref, ei1_ref, ad_ref,
                 W1s, cW2s, W2es, aW2s, b2s,
                 cW1t, cb1t, cgt, cbbt, cW2t, cb2t,
                 eW1t, eb1t, egt, ebbt, eW2t, eb2t,
                 aW1t, ab1t, agt, abbt, aW2t, ab2t,
                 sems):
    i = pl.program_id(0)
    L = _LANES
    D = x_ref.shape[1]
    f32 = jnp.float32
    bf16 = jnp.bfloat16

    @pl.when(i == 0)
    def _():
        srcs = (cW1_h, cb1_h, cg_h, cbb_h, cW2_h, cb2_h,
                eW1_h, eb1_h, eg_h, ebb_h, eW2_h, eb2_h,
                aW1_h, ab1_h, ag_h, abb_h, aW2_h, ab2_h)
        dsts = (cW1t, cb1t, cgt, cbbt, cW2t, cb2t,
                eW1t, eb1t, egt, ebbt, eW2t, eb2t,
                aW1t, ab1t, agt, abbt, aW2t, ab2t)
        copies = [pltpu.make_async_copy(s, d, sems.at[k])
                  for k, (s, d) in enumerate(zip(srcs, dsts))]
        for cp in copies:
            cp.start()
        for cp in copies:
            cp.wait()

        W1s[:, 0:H] = cW1t[...].astype(bf16)
        for c in range(C):
            W1s[:, H + c * H:H + (c + 1) * H] = eW1t[c * D:(c + 1) * D, :].astype(bf16)
        W1s[:, H + C * H:H + C * H + AH] = aW1t[...].astype(bf16)

        nc = cW2t.shape[0]
        cW2s[...] = jnp.concatenate(
            [jnp.transpose(cW2t[...], (1, 0)).astype(bf16),
             jnp.zeros((H, L - nc), bf16)], axis=1)
        ne = eW2t.shape[0] // C
        for c in range(C):
            parts = []
            if c > 0:
                parts.append(jnp.zeros((H, _GRP * c), bf16))
            parts.append(jnp.transpose(eW2t[c * ne:(c + 1) * ne, :], (1, 0)).astype(bf16))
            parts.append(jnp.zeros((H, L - _GRP * c - ne), bf16))
            W2es[c * H:(c + 1) * H, :] = jnp.concatenate(parts, axis=1)
        aW2s[...] = jnp.concatenate(
            [jnp.transpose(aW2t[...], (1, 0)).astype(bf16),
             jnp.zeros((AH, L - 1), bf16)], axis=1)

        b2s[0:1, :] = jnp.concatenate(
            [cb2t[...], jnp.full((1, L - nc), _NEG, f32)], axis=1)
        erow = jnp.full((1, L), _NEG, f32)
        lane = jax.lax.broadcasted_iota(jnp.int32, (1, L), 1)
        for c in range(C):
            parts = []
            if c > 0:
                parts.append(jnp.zeros((1, _GRP * c), f32))
            parts.append(eb2t[c:c + 1, :])
            parts.append(jnp.zeros((1, L - _GRP * c - ne), f32))
            row = jnp.concatenate(parts, axis=1)
            m = (lane >= _GRP * c) & (lane < _GRP * c + ne)
            erow = jnp.where(m, row, erow)
        b2s[1:2, :] = erow
        b2s[2:3, :] = jnp.concatenate(
            [ab2t[...], jnp.zeros((1, L - 1), f32)], axis=1)

    xb = x_ref[...].astype(bf16)

    mm = jnp.dot(xb, W1s[...], preferred_element_type=jnp.float32)

    h = _gelu(_ln_f32(mm[:, :H] + cb1t[...], cgt[...], cbbt[...]))
    cl = jnp.dot(h.astype(jnp.bfloat16), cW2s[...],
                 preferred_element_type=jnp.float32) + b2s[0:1, :]
    clT = jnp.transpose(cl, (1, 0))[:C]
    cp_ = _softmax_t(clT)
    cw1, ci1, cw2, ci2 = _top2_t(cp_)

    ehb = []
    for c in range(C):
        seg = mm[:, H + c * H:H + (c + 1) * H]
        eh = _gelu(_ln_f32(seg + eb1t[c:c + 1, :], egt[c:c + 1, :],
                           ebbt[c:c + 1, :]))
        ehb.append(eh.astype(jnp.bfloat16))
    ehb = jnp.concatenate(ehb, axis=1)
    el = jnp.dot(ehb, W2es[...],
                 preferred_element_type=jnp.float32) + b2s[1:2, :]
    elT = jnp.transpose(el, (1, 0))

    sel0 = jnp.full((_GRP, elT.shape[1]), _NEG, dtype=jnp.float32)
    sel1 = jnp.full((_GRP, elT.shape[1]), _NEG, dtype=jnp.float32)
    for c in range(C):
        grp = elT[c * _GRP:(c + 1) * _GRP]
        sel0 = jnp.where(ci1 == c, grp, sel0)
        sel1 = jnp.where(ci2 == c, grp, sel1)
    e0w1, e0i1, e0w2, e0i2 = _top2_t(_softmax_t(sel0))
    e1w1, e1i1, e1w2, e1i2 = _top2_t(_softmax_t(sel1))

    ah = _gelu(_ln_f32(mm[:, H + C * H:H + C * H + AH] + ab1t[...],
                       agt[...], abbt[...]))
    av = jnp.dot(ah.astype(jnp.bfloat16), aW2s[...],
                 preferred_element_type=jnp.float32) + b2s[2:3, :]
    ad = jax.nn.sigmoid(av[:, 0:1])

    cw_ref[...] = jnp.concatenate([cw1, cw2], axis=0)
    ci_ref[...] = jnp.concatenate([ci1, ci2], axis=0).astype(jnp.int32)
    ew0_ref[...] = jnp.concatenate([e0w1, e0w2], axis=0)
    ei0_ref[...] = jnp.concatenate([e0i1, e0i2], axis=0).astype(jnp.int32)
    ew1_ref[...] = jnp.concatenate([e1w1, e1w2], axis=0)
    ei1_ref[...] = jnp.concatenate([e1i1, e1i2], axis=0).astype(jnp.int32)
    ad_ref[...] = jnp.transpose(ad, (1, 0))


def kernel(x, cW1, cb1, cg, cbb, cW2, cb2, eW1, eb1, eg, ebb, eW2, eb2,
           aW1, ab1, ag, abb, aW2, ab2):
    Bz, Sz, Dz = x.shape
    T = Bz * Sz
    C = eW1.shape[0]
    H = cW1.shape[1]
    AH = aW1.shape[1]
    EPC = eW2.shape[2]
    L = _LANES

    xf = x.reshape(T, Dz)
    eW1f = eW1.reshape(C * Dz, H)
    eW2f = eW2.transpose(0, 2, 1).reshape(C * EPC, H)
    cb1r, cgr, cbbr = cb1.reshape(1, H), cg.reshape(1, H), cbb.reshape(1, H)
    ab1r, agr, abbr = ab1.reshape(1, AH), ag.reshape(1, AH), abb.reshape(1, AH)
    cW2r = cW2.T
    aW2r = aW2.T
    cb2r = cb2.reshape(1, C)
    ab2r = ab2.reshape(1, 1)

    grid = (T // _TM,)
    tok_spec = pl.BlockSpec((_TM, Dz), lambda i: (i, 0))
    anyspec = pl.BlockSpec(memory_space=pl.ANY)
    out2 = pl.BlockSpec((2, _TM), lambda i: (0, i))
    out1 = pl.BlockSpec((1, _TM), lambda i: (0, i))

    f32 = jnp.float32
    i32 = jnp.int32
    outs = (
        jax.ShapeDtypeStruct((2, T), f32), jax.ShapeDtypeStruct((2, T), i32),
        jax.ShapeDtypeStruct((2, T), f32), jax.ShapeDtypeStruct((2, T), i32),
        jax.ShapeDtypeStruct((2, T), f32), jax.ShapeDtypeStruct((2, T), i32),
        jax.ShapeDtypeStruct((1, T), f32),
    )
    V = pltpu.VMEM

    cw, ci, ew0, ei0, ew1, ei1, ad = pl.pallas_call(
        functools.partial(_router_body, C, H, AH),
        grid=grid,
        in_specs=[tok_spec] + [anyspec] * 18,
        out_specs=[out2, out2, out2, out2, out2, out2, out1],
        out_shape=outs,
        scratch_shapes=[
            V((Dz, H + C * H + AH), jnp.bfloat16),
            V((H, L), jnp.bfloat16),
            V((C * H, L), jnp.bfloat16),
            V((AH, L), jnp.bfloat16),
            V((8, L), f32),
            V((Dz, H), f32), V((1, H), f32), V((1, H), f32), V((1, H), f32),
            V((C, H), f32), V((1, C), f32),
            V((C * Dz, H), f32), V((C, H), f32), V((C, H), f32), V((C, H), f32),
            V((C * EPC, H), f32), V((C, EPC), f32),
            V((Dz, AH), f32), V((1, AH), f32), V((1, AH), f32), V((1, AH), f32),
            V((1, AH), f32), V((1, 1), f32),
            pltpu.SemaphoreType.DMA((18,)),
        ],
        compiler_params=pltpu.CompilerParams(
            dimension_semantics=("arbitrary",),
        ),
    )(xf,
      cW1, cb1r, cgr, cbbr, cW2r, cb2r,
      eW1f, eb1, eg, ebb, eW2f, eb2,
      aW1, ab1r, agr, abbr, aW2r, ab2r)

    return (cw.T, ci.T, ew0.T, ei0.T, ew1.T, ei1.T, ad.reshape(Bz, Sz, 1))
